# baseline (speedup 1.0000x reference)
import jax
import jax.numpy as jnp
import numpy as np
from jax.experimental import pallas as pl
from jax.experimental.pallas import tpu as pltpu

E = 8
D = 1024
H = 256
H2 = 2 * H
N = 8192
EH = E * H2
LANES = 128
NEG = -1e30

T = 512

_PAD = ((0, 0), (0, LANES - E))


def _noise(seed):
    noise = jax.random.normal(jax.random.key(seed), (N, E), jnp.float32)
    return jnp.pad(noise, _PAD, constant_values=NEG)


try:
    NOISE1 = np.asarray(_noise(123))
    NOISE2 = np.asarray(_noise(456))
except Exception:
    NOISE1, NOISE2 = None, None


def _noise_operands():
    if NOISE1 is not None:
        return jnp.asarray(NOISE1), jnp.asarray(NOISE2)
    return _noise(123), _noise(456)


def _top2_weights(gates):
    col = jax.lax.broadcasted_iota(jnp.int32, gates.shape, 1)
    m1 = jnp.max(gates, axis=1, keepdims=True)
    i1 = jnp.min(jnp.where(gates == m1, col, LANES), axis=1, keepdims=True)
    g2 = jnp.where(col == i1, -1.0, gates)
    m2 = jnp.max(g2, axis=1, keepdims=True)
    i2 = jnp.min(jnp.where(g2 == m2, col, LANES), axis=1, keepdims=True)
    return jnp.where(col == i1, m1, 0.0) + jnp.where(col == i2, m2, 0.0)


def _moe_body(x1_ref, n1_ref, g1_ref, gb1_ref, w11_any, b11_ref, w21_any, b21_ref,
              x2_ref, n2_ref, g2_ref, gb2_ref, w12_any, b12_ref, w22_any, b22_ref,
              wc_ref, bc_ref, out_ref,
              w1s1, w2s1, w1s2, w2s2, gp1, gp2, gbp1, gbp2, hws, stg1, stg2, sem):

    @pl.when(pl.program_id(0) == 0)
    def _load_weights():
        for g_ref, gb_ref, gp, gbp in ((g1_ref, gb1_ref, gp1, gbp1),
                                       (g2_ref, gb2_ref, gp2, gbp2)):
            gp[...] = jnp.zeros((D, LANES), jnp.float32)
            gp[:, :E] = g_ref[...]
            gbp[...] = jnp.zeros((1, LANES), jnp.float32)
            gbp[:, :E] = gb_ref[...]
        for w1_any, w2_any, w1s, w2s in (
                (w11_any, w21_any, w1s1, w2s1),
                (w12_any, w22_any, w1s2, w2s2)):
            for e in range(E):
                pltpu.make_async_copy(w1_any.at[e], stg1, sem).start()
                pltpu.make_async_copy(w1_any.at[e], stg1, sem).wait()
                w1s[:, e * H2:(e + 1) * H2] = stg1[...].astype(jnp.bfloat16)
                pltpu.make_async_copy(w2_any.at[e], stg2, sem).start()
                pltpu.make_async_copy(w2_any.at[e], stg2, sem).wait()
                w2s[e * H2:(e + 1) * H2, :] = stg2[...].astype(jnp.bfloat16)

    def branch(x_ref, n_ref, gp, gbp, w1s, b1_ref, w2s, b2_ref):
        x = x_ref[...]
        logits = jnp.dot(x, gp[...], preferred_element_type=jnp.float32)
        z = logits + gbp[...] + n_ref[...]
        zmax = jnp.max(z, axis=1, keepdims=True)
        ez = jnp.exp(z - zmax)
        gates = ez / jnp.sum(ez, axis=1, keepdims=True)
        w = _top2_weights(gates)
        w16 = w.astype(jnp.bfloat16)
        xb = x.astype(jnp.bfloat16)
        h = jnp.dot(xb, w1s[...],
                    preferred_element_type=jnp.float32).astype(jnp.bfloat16)
        b1_16 = b1_ref[...].astype(jnp.bfloat16)
        ob = None
        for e in range(E):
            we = w[:, e:e + 1]
            he = jnp.maximum(h[:, e * H2:(e + 1) * H2] + b1_16[e], 0.0)
            hws[:, e * H2:(e + 1) * H2] = he * w16[:, e:e + 1]
            obe = we * b2_ref[e]
            ob = obe if ob is None else ob + obe
        o = jnp.dot(hws[...], w2s[...], preferred_element_type=jnp.float32)
        return o + ob

    m1 = branch(x1_ref, n1_ref, gp1, gbp1, w1s1, b11_ref, w2s1, b21_ref)
    m2 = branch(x2_ref, n2_ref, gp2, gbp2, w1s2, b12_ref, w2s2, b22_ref)
    mcat = jnp.concatenate([m1, m2], axis=1)
    out_ref[...] = jnp.dot(mcat, wc_ref[...],
                           preferred_element_type=jnp.float32) + bc_ref[...]


def kernel(x1, x2, W1_1, b1_1, W2_1, b2_1, W1_2, b1_2, W2_2, b2_2,
           G1, gb1, G2, gb2, Wc, bc):
    n1, n2 = _noise_operands()
    gb1r = gb1.reshape(1, E)
    gb2r = gb2.reshape(1, E)
    bcr = bc.reshape(1, 1)

    tile = lambda i: (i, 0)
    whole2 = lambda s: pl.BlockSpec(s, lambda i: (0, 0))
    anyspec = pl.BlockSpec(memory_space=pl.ANY)

    out = pl.pallas_call(
        _moe_body,
        grid=(N // T,),
        in_specs=[
            pl.BlockSpec((T, D), tile),
            pl.BlockSpec((T, LANES), tile),
            whole2((D, E)),
            whole2((1, E)),
            anyspec,
            whole2((E, H2)),
            anyspec,
            whole2((E, H)),
            pl.BlockSpec((T, D), tile),
            pl.BlockSpec((T, LANES), tile),
            whole2((D, E)),
            whole2((1, E)),
            anyspec,
            whole2((E, H2)),
            anyspec,
            whole2((E, H)),
            whole2((H2, 1)),
            whole2((1, 1)),
        ],
        out_specs=pl.BlockSpec((T, 1), tile),
        out_shape=jax.ShapeDtypeStruct((N, 1), jnp.float32),
        scratch_shapes=[
            pltpu.VMEM((D, EH), jnp.bfloat16),
            pltpu.VMEM((EH, H), jnp.bfloat16),
            pltpu.VMEM((D, EH), jnp.bfloat16),
            pltpu.VMEM((EH, H), jnp.bfloat16),
            pltpu.VMEM((D, LANES), jnp.float32),
            pltpu.VMEM((D, LANES), jnp.float32),
            pltpu.VMEM((1, LANES), jnp.float32),
            pltpu.VMEM((1, LANES), jnp.float32),
            pltpu.VMEM((T, EH), jnp.bfloat16),
            pltpu.VMEM((D, H2), jnp.float32),
            pltpu.VMEM((H2, H), jnp.float32),
            pltpu.SemaphoreType.DMA,
        ],
    )(x1, n1, G1, gb1r, W1_1, b1_1, W2_1, b2_1,
      x2, n2, G2, gb2r, W1_2, b1_2, W2_2, b2_2,
      Wc, bcr)

    return out


# speedup vs baseline: 1.0622x; 1.0622x over previous
import jax
import jax.numpy as jnp
import numpy as np
from jax.experimental import pallas as pl
from jax.experimental.pallas import tpu as pltpu

E = 8
D = 1024
H = 256
H2 = 2 * H
N = 8192
EH = E * H2
LANES = 128
NEG = -1e30

T = 512

_PAD = ((0, 0), (0, LANES - E))


def _noise(seed):
    noise = jax.random.normal(jax.random.key(seed), (N, E), jnp.float32)
    return jnp.pad(noise, _PAD, constant_values=NEG)


try:
    NOISE1 = np.asarray(_noise(123))
    NOISE2 = np.asarray(_noise(456))
except Exception:
    NOISE1, NOISE2 = None, None


def _noise_operands():
    if NOISE1 is not None:
        return jnp.asarray(NOISE1), jnp.asarray(NOISE2)
    return _noise(123), _noise(456)


def _top2_weights(gates):
    col = jax.lax.broadcasted_iota(jnp.int32, gates.shape, 1)
    m1 = jnp.max(gates, axis=1, keepdims=True)
    i1 = jnp.min(jnp.where(gates == m1, col, LANES), axis=1, keepdims=True)
    g2 = jnp.where(col == i1, -1.0, gates)
    m2 = jnp.max(g2, axis=1, keepdims=True)
    i2 = jnp.min(jnp.where(g2 == m2, col, LANES), axis=1, keepdims=True)
    return jnp.where(col == i1, m1, 0.0) + jnp.where(col == i2, m2, 0.0)


def _moe_body(x1_ref, n1_ref, g1_ref, gb1_ref, w11_any, b11_ref, w21_any, b21_ref,
              x2_ref, n2_ref, g2_ref, gb2_ref, w12_any, b12_ref, w22_any, b22_ref,
              wc_ref, bc_ref, out_ref,
              w1s1, w2s1, w1s2, w2s2, gp1, gp2, gbp1, gbp2, stg1, stg2,
              sem1, sem2):

    @pl.when(pl.program_id(0) == 0)
    def _load_weights():
        for g_ref, gb_ref, gp, gbp in ((g1_ref, gb1_ref, gp1, gbp1),
                                       (g2_ref, gb2_ref, gp2, gbp2)):
            gp[...] = jnp.zeros((D, LANES), jnp.float32)
            gp[:, :E] = g_ref[...]
            gbp[...] = jnp.zeros((1, LANES), jnp.float32)
            gbp[:, :E] = gb_ref[...]
        plan = []
        for w1_any, w2_any, w1s, w2s in (
                (w11_any, w21_any, w1s1, w2s1),
                (w12_any, w22_any, w1s2, w2s2)):
            for e in range(E):
                plan.append((w1_any.at[e], stg1, sem1,
                             lambda w1s=w1s, e=e, s=stg1:
                             w1s.__setitem__((slice(None), slice(e * H2, (e + 1) * H2)),
                                             s[...].astype(jnp.bfloat16))))
                plan.append((w2_any.at[e], stg2, sem2,
                             lambda w2s=w2s, e=e, s=stg2:
                             w2s.__setitem__((slice(e * H2, (e + 1) * H2), slice(None)),
                                             s[...].astype(jnp.bfloat16))))
        pltpu.make_async_copy(*plan[0][:3]).start()
        for i, (src, dst, sem, cast) in enumerate(plan):
            if i + 1 < len(plan):
                nsrc, ndst, nsem, _ = plan[i + 1]
                pltpu.make_async_copy(nsrc, ndst, nsem).start()
            pltpu.make_async_copy(src, dst, sem).wait()
            cast()

    def branch(x_ref, n_ref, gp, gbp, w1s, b1_ref, w2s, b2_ref):
        x = x_ref[...]
        logits = jnp.dot(x, gp[...], preferred_element_type=jnp.float32)
        z = logits + gbp[...] + n_ref[...]
        zmax = jnp.max(z, axis=1, keepdims=True)
        ez = jnp.exp(z - zmax)
        gates = ez / jnp.sum(ez, axis=1, keepdims=True)
        w = _top2_weights(gates)
        w16 = w.astype(jnp.bfloat16)
        xb = x.astype(jnp.bfloat16)
        h = jnp.dot(xb, w1s[...],
                    preferred_element_type=jnp.float32).astype(jnp.bfloat16)
        b1_16 = b1_ref[...].astype(jnp.bfloat16)
        ob = None
        parts = []
        for e in range(E):
            we = w[:, e:e + 1]
            he = jnp.maximum(h[:, e * H2:(e + 1) * H2] + b1_16[e], 0.0)
            parts.append(he * w16[:, e:e + 1])
            obe = we * b2_ref[e]
            ob = obe if ob is None else ob + obe
        hw = jnp.concatenate(parts, axis=1)
        o = jnp.dot(hw, w2s[...], preferred_element_type=jnp.float32)
        return o + ob

    m1 = branch(x1_ref, n1_ref, gp1, gbp1, w1s1, b11_ref, w2s1, b21_ref)
    m2 = branch(x2_ref, n2_ref, gp2, gbp2, w1s2, b12_ref, w2s2, b22_ref)
    mcat = jnp.concatenate([m1, m2], axis=1)
    out_ref[...] = jnp.dot(mcat, wc_ref[...],
                           preferred_element_type=jnp.float32) + bc_ref[...]


def kernel(x1, x2, W1_1, b1_1, W2_1, b2_1, W1_2, b1_2, W2_2, b2_2,
           G1, gb1, G2, gb2, Wc, bc):
    n1, n2 = _noise_operands()
    gb1r = gb1.reshape(1, E)
    gb2r = gb2.reshape(1, E)
    bcr = bc.reshape(1, 1)

    tile = lambda i: (i, 0)
    whole2 = lambda s: pl.BlockSpec(s, lambda i: (0, 0))
    anyspec = pl.BlockSpec(memory_space=pl.ANY)

    out = pl.pallas_call(
        _moe_body,
        grid=(N // T,),
        in_specs=[
            pl.BlockSpec((T, D), tile),
            pl.BlockSpec((T, LANES), tile),
            whole2((D, E)),
            whole2((1, E)),
            anyspec,
            whole2((E, H2)),
            anyspec,
            whole2((E, H)),
            pl.BlockSpec((T, D), tile),
            pl.BlockSpec((T, LANES), tile),
            whole2((D, E)),
            whole2((1, E)),
            anyspec,
            whole2((E, H2)),
            anyspec,
            whole2((E, H)),
            whole2((H2, 1)),
            whole2((1, 1)),
        ],
        out_specs=pl.BlockSpec((T, 1), tile),
        out_shape=jax.ShapeDtypeStruct((N, 1), jnp.float32),
        scratch_shapes=[
            pltpu.VMEM((D, EH), jnp.bfloat16),
            pltpu.VMEM((EH, H), jnp.bfloat16),
            pltpu.VMEM((D, EH), jnp.bfloat16),
            pltpu.VMEM((EH, H), jnp.bfloat16),
            pltpu.VMEM((D, LANES), jnp.float32),
            pltpu.VMEM((D, LANES), jnp.float32),
            pltpu.VMEM((1, LANES), jnp.float32),
            pltpu.VMEM((1, LANES), jnp.float32),
            pltpu.VMEM((D, H2), jnp.float32),
            pltpu.VMEM((H2, H), jnp.float32),
            pltpu.SemaphoreType.DMA,
            pltpu.SemaphoreType.DMA,
        ],
    )(x1, n1, G1, gb1r, W1_1, b1_1, W2_1, b2_1,
      x2, n2, G2, gb2r, W1_2, b1_2, W2_2, b2_2,
      Wc, bcr)

    return out
